# baseline (speedup 1.0000x reference)
import functools

import jax
import jax.numpy as jnp
from jax import lax
from jax.experimental import pallas as pl
from jax.experimental.pallas import tpu as pltpu
from jax.experimental.pallas import tpu_sc as plsc

N = 100000
K = 16
C_IN = 16
C_OUT = 64
C_MID = 16
ROW = 32

NK = N * K
NUM_WORKERS = 32
PER_W = NK // NUM_WORKERS
CHUNK = 1000
NCHUNK = PER_W // CHUNK

B = 1000
GRID = N // B


def _sc_gather(table, idx_flat):
    mesh = plsc.VectorSubcoreMesh(core_axis_name="c", subcore_axis_name="s")

    @functools.partial(
        pl.kernel,
        mesh=mesh,
        out_type=jax.ShapeDtypeStruct((NK, ROW), jnp.bfloat16),
        scratch_types=[
            pltpu.VMEM((CHUNK,), jnp.int32),
            pltpu.VMEM((CHUNK,), jnp.int32),
            pltpu.VMEM((CHUNK, ROW), jnp.bfloat16),
            pltpu.VMEM((CHUNK, ROW), jnp.bfloat16),
            pltpu.SemaphoreType.DMA,
            pltpu.SemaphoreType.DMA,
            pltpu.SemaphoreType.DMA,
            pltpu.SemaphoreType.DMA,
        ],
        compiler_params=pltpu.CompilerParams(use_tc_tiling_on_sc=False),
    )
    def gather_kernel(table_hbm, idx_hbm, out_hbm,
                      idx0, idx1, rows0, rows1, gsem0, gsem1, wsem0, wsem1):
        wid = lax.axis_index("s") * 2 + lax.axis_index("c")
        base = pl.multiple_of(wid * PER_W, 8)
        idx_b = (idx0, idx1)
        rows_b = (rows0, rows1)
        gsem_b = (gsem0, gsem1)
        wsem_b = (wsem0, wsem1)

        for b in range(2):
            pltpu.sync_copy(idx_hbm.at[pl.ds(base + b * CHUNK, CHUNK)],
                            idx_b[b])
            pltpu.async_copy(table_hbm.at[idx_b[b]], rows_b[b], gsem_b[b])

        def body(i, carry):
            for b in range(2):
                j = 2 * i + b
                off = pl.multiple_of(base + j * CHUNK, 8)
                pltpu.make_async_copy(table_hbm.at[idx_b[b]], rows_b[b],
                                      gsem_b[b]).wait()
                pltpu.async_copy(rows_b[b], out_hbm.at[pl.ds(off, CHUNK)],
                                 wsem_b[b])

                @pl.when(j + 2 < NCHUNK)
                def _prefetch():
                    off2 = pl.multiple_of(off + 2 * CHUNK, 8)
                    pltpu.sync_copy(idx_hbm.at[pl.ds(off2, CHUNK)], idx_b[b])
                    pltpu.make_async_copy(rows_b[b],
                                          out_hbm.at[pl.ds(off, CHUNK)],
                                          wsem_b[b]).wait()
                    pltpu.async_copy(table_hbm.at[idx_b[b]], rows_b[b],
                                     gsem_b[b])

                @pl.when(j + 2 >= NCHUNK)
                def _drain():
                    pltpu.make_async_copy(rows_b[b],
                                          out_hbm.at[pl.ds(off, CHUNK)],
                                          wsem_b[b]).wait()
            return carry

        lax.fori_loop(0, NCHUNK // 2, body, 0)

    return gather_kernel(table, idx_flat)


def _tc_body(xyz_ref, g_ref,
             d_ref, t_ref, w1_ref, b1_ref, w2_ref, b2_ref, w3_ref, b3_ref,
             e_ref, f_ref, lwt_ref, lb_ref,
             local_ref, out_ref):
    x = xyz_ref[...]
    g = g_ref[...]
    f32 = jnp.float32
    local = (jnp.dot(g, d_ref[...], preferred_element_type=f32)
             - jnp.dot(x, t_ref[...], preferred_element_type=f32))
    local_ref[...] = local

    h1 = jnp.maximum(jnp.dot(local, w1_ref[...], preferred_element_type=f32)
                     + b1_ref[...], 0.0)
    h2 = jnp.dot(h1, w2_ref[...], preferred_element_type=f32) + b2_ref[...]
    w = jnp.maximum(jnp.dot(h2, w3_ref[...], preferred_element_type=f32)
                    + b3_ref[...], 0.0)
    w_bf = w.astype(jnp.bfloat16)

    e = e_ref[...]
    f = f_ref[...]
    pre = jnp.zeros((B, C_IN * C_MID), f32)
    for k in range(K):
        gfk = g[:, k * ROW:k * ROW + C_IN]
        wk = w_bf[:, k * C_MID:(k + 1) * C_MID]
        pre = pre + (jnp.dot(gfk, e, preferred_element_type=f32)
                     * jnp.dot(wk, f, preferred_element_type=f32))

    out = jnp.dot(pre, lwt_ref[...], preferred_element_type=f32) + lb_ref[...]
    out_ref[...] = jnp.maximum(out, 0.0)


def _tc_dense(xyz, g, d, t, w1b, b1b, w2b, b2b, w3b, b3b, e, f, lwt, lb2):
    full = lambda shape: pl.BlockSpec(shape, lambda i: (0, 0))
    row = lambda width: pl.BlockSpec((B, width), lambda i: (i, 0))
    return pl.pallas_call(
        _tc_body,
        grid=(GRID,),
        in_specs=[
            row(3), row(ROW * K),
            full((ROW * K, 3 * K)), full((3, 3 * K)),
            full((3 * K, 8 * K)), full((1, 8 * K)),
            full((8 * K, 8 * K)), full((1, 8 * K)),
            full((8 * K, C_MID * K)), full((1, C_MID * K)),
            full((C_IN, C_IN * C_MID)), full((C_MID, C_IN * C_MID)),
            full((C_IN * C_MID, C_OUT)), full((1, C_OUT)),
        ],
        out_specs=[
            pl.BlockSpec((B, 3 * K), lambda i: (i, 0)),
            pl.BlockSpec((B, C_OUT), lambda i: (i, 0)),
        ],
        out_shape=[
            jax.ShapeDtypeStruct((N, 3 * K), jnp.float32),
            jax.ShapeDtypeStruct((N, C_OUT), jnp.float32),
        ],
    )(xyz, g, d, t, w1b, b1b, w2b, b2b, w3b, b3b, e, f, lwt, lb2)


def kernel(dense_xyz, dense_feats, nei_inds,
           w1, b1, g1, be1, w2, b2, g2, be2, w3, b3, g3, be3, lw, lb):
    xyz = dense_xyz[0]
    feats = dense_feats[0]
    idx_flat = nei_inds[0].reshape(-1).astype(jnp.int32)
    table = jnp.concatenate(
        [feats, xyz, jnp.zeros((N, ROW - C_IN - 3), jnp.float32)],
        axis=1).astype(jnp.bfloat16)

    grows = _sc_gather(table, idx_flat)
    g2d = grows.reshape(N, K * ROW)

    inv = 1.0 / jnp.sqrt(1.0 + 1e-5)
    s1, s2, s3 = g1 * inv, g2 * inv, g3 * inv
    w1e = w1.T * s1[None, :]
    c1 = b1 * s1 + be1
    w2e = w2.T * s2[None, :]
    c2 = b2 * s2 + be2
    w3e = w3.T * s3[None, :]
    c3 = b3 * s3 + be3

    eyeK = jnp.eye(K, dtype=jnp.float32)
    w1b = jnp.kron(eyeK, w1e)
    w2b = jnp.kron(eyeK, w2e)
    w3b = jnp.kron(eyeK, w3e)
    b1b = jnp.tile(c1, K)[None, :]
    b2b = jnp.tile(c2, K)[None, :]
    b3b = jnp.tile(c3, K)[None, :]

    dk = jnp.zeros((ROW, 3), jnp.float32).at[C_IN:C_IN + 3, :].set(
        jnp.eye(3, dtype=jnp.float32))
    d = jnp.kron(eyeK, dk).astype(jnp.bfloat16)
    t = jnp.tile(jnp.eye(3, dtype=jnp.float32), (1, K))
    e = jnp.kron(jnp.eye(C_IN, dtype=jnp.float32),
                 jnp.ones((1, C_MID), jnp.float32)
                 ).astype(jnp.bfloat16)
    f = jnp.tile(jnp.eye(C_MID, dtype=jnp.float32),
                 (1, C_IN)).astype(jnp.bfloat16)

    local48, new_feat = _tc_dense(xyz, g2d, d, t,
                                  w1b, b1b, w2b, b2b, w3b, b3b,
                                  e, f, lw.T, lb[None, :])
    return (new_feat[None], local48.reshape(1, N, K, 3))


# speedup vs baseline: 1.2222x; 1.2222x over previous
import functools

import jax
import jax.numpy as jnp
from jax import lax
from jax.experimental import pallas as pl
from jax.experimental.pallas import tpu as pltpu
from jax.experimental.pallas import tpu_sc as plsc

N = 100000
K = 16
C_IN = 16
C_OUT = 64
C_MID = 16
ROW = 32

NK = N * K
NUM_WORKERS = 32
PER_W = NK // NUM_WORKERS
CHUNK = 1000
NCHUNK = PER_W // CHUNK

B = 1000
GRID = N // B


def _sc_gather(table, idx_flat):
    mesh = plsc.VectorSubcoreMesh(core_axis_name="c", subcore_axis_name="s")

    @functools.partial(
        pl.kernel,
        mesh=mesh,
        out_type=jax.ShapeDtypeStruct((NK, ROW), jnp.float32),
        scratch_types=[
            pltpu.VMEM((CHUNK,), jnp.int32),
            pltpu.VMEM((CHUNK,), jnp.int32),
            pltpu.VMEM((CHUNK, ROW), jnp.float32),
            pltpu.VMEM((CHUNK, ROW), jnp.float32),
            pltpu.SemaphoreType.DMA,
            pltpu.SemaphoreType.DMA,
            pltpu.SemaphoreType.DMA,
            pltpu.SemaphoreType.DMA,
        ],
        compiler_params=pltpu.CompilerParams(use_tc_tiling_on_sc=False),
    )
    def gather_kernel(table_hbm, idx_hbm, out_hbm,
                      idx0, idx1, rows0, rows1, gsem0, gsem1, wsem0, wsem1):
        wid = lax.axis_index("s") * 2 + lax.axis_index("c")
        base = pl.multiple_of(wid * PER_W, 8)
        idx_b = (idx0, idx1)
        rows_b = (rows0, rows1)
        gsem_b = (gsem0, gsem1)
        wsem_b = (wsem0, wsem1)

        for b in range(2):
            pltpu.sync_copy(idx_hbm.at[pl.ds(base + b * CHUNK, CHUNK)],
                            idx_b[b])
            pltpu.async_copy(table_hbm.at[idx_b[b]], rows_b[b], gsem_b[b])

        def body(i, carry):
            for b in range(2):
                j = 2 * i + b
                off = pl.multiple_of(base + j * CHUNK, 8)
                dst = out_hbm.at[pl.ds(off, CHUNK)]
                pltpu.make_async_copy(table_hbm.at[idx_b[b]], rows_b[b],
                                      gsem_b[b]).wait()
                pltpu.async_copy(rows_b[b], dst, wsem_b[b])

                @pl.when(j + 2 < NCHUNK)
                def _prefetch():
                    off2 = pl.multiple_of(off + 2 * CHUNK, 8)
                    pltpu.sync_copy(idx_hbm.at[pl.ds(off2, CHUNK)], idx_b[b])
                    pltpu.make_async_copy(rows_b[b], dst, wsem_b[b]).wait()
                    pltpu.async_copy(table_hbm.at[idx_b[b]], rows_b[b],
                                     gsem_b[b])

                @pl.when(j + 2 >= NCHUNK)
                def _drain():
                    pltpu.make_async_copy(rows_b[b], dst, wsem_b[b]).wait()
            return carry

        lax.fori_loop(0, NCHUNK // 2, body, 0)

    return gather_kernel(table, idx_flat)


def _tc_body(xyz_ref, g0_ref, g1_ref, g2_ref, g3_ref,
             d_ref, t_ref, w1_ref, b1_ref, w2_ref, b2_ref, w3_ref, b3_ref,
             e_ref, f_ref, lwt_ref, lb_ref,
             local_ref, out_ref):
    x = xyz_ref[...]
    f32 = jnp.float32
    qs = [r[:, 0, 0, :] for r in (g0_ref, g1_ref, g2_ref, g3_ref)]
    acc = -jnp.dot(x, t_ref[...], preferred_element_type=f32)
    for q in range(4):
        acc = acc + jnp.dot(qs[q], d_ref[q * 128:(q + 1) * 128, :],
                            preferred_element_type=f32)
    local = acc
    local_ref[...] = local

    h1 = jnp.maximum(jnp.dot(local, w1_ref[...], preferred_element_type=f32)
                     + b1_ref[...], 0.0)
    h2 = jnp.dot(h1, w2_ref[...], preferred_element_type=f32) + b2_ref[...]
    w = jnp.maximum(jnp.dot(h2, w3_ref[...], preferred_element_type=f32)
                    + b3_ref[...], 0.0)
    w_bf = w.astype(jnp.bfloat16)
    qs_bf = [q.astype(jnp.bfloat16) for q in qs]

    e = e_ref[...]
    f = f_ref[...]
    pre = jnp.zeros((B, C_IN * C_MID), f32)
    for k in range(K):
        gfk = qs_bf[k // 4][:, (k % 4) * ROW:(k % 4) * ROW + C_IN]
        wk = w_bf[:, k * C_MID:(k + 1) * C_MID]
        pre = pre + (jnp.dot(gfk, e, preferred_element_type=f32)
                     * jnp.dot(wk, f, preferred_element_type=f32))

    out = jnp.dot(pre, lwt_ref[...], preferred_element_type=f32) + lb_ref[...]
    out_ref[...] = jnp.maximum(out, 0.0)


def _tc_dense(xyz, g, d, t, w1b, b1b, w2b, b2b, w3b, b3b, e, f, lwt, lb2):
    full = lambda shape: pl.BlockSpec(shape, lambda i: (0, 0))
    row = lambda width: pl.BlockSpec((B, width), lambda i: (i, 0))
    return pl.pallas_call(
        _tc_body,
        grid=(GRID,),
        in_specs=[
            row(3),
            pl.BlockSpec((B, 1, 1, 128), lambda i: (i, 0, 0, 0)),
            pl.BlockSpec((B, 1, 1, 128), lambda i: (i, 1, 0, 0)),
            pl.BlockSpec((B, 1, 1, 128), lambda i: (i, 2, 0, 0)),
            pl.BlockSpec((B, 1, 1, 128), lambda i: (i, 3, 0, 0)),
            full((ROW * K, 3 * K)), full((3, 3 * K)),
            full((3 * K, 8 * K)), full((1, 8 * K)),
            full((8 * K, 8 * K)), full((1, 8 * K)),
            full((8 * K, C_MID * K)), full((1, C_MID * K)),
            full((C_IN, C_IN * C_MID)), full((C_MID, C_IN * C_MID)),
            full((C_IN * C_MID, C_OUT)), full((1, C_OUT)),
        ],
        out_specs=[
            pl.BlockSpec((B, 3 * K), lambda i: (i, 0)),
            pl.BlockSpec((B, C_OUT), lambda i: (i, 0)),
        ],
        out_shape=[
            jax.ShapeDtypeStruct((N, 3 * K), jnp.float32),
            jax.ShapeDtypeStruct((N, C_OUT), jnp.float32),
        ],
    )(xyz, g, g, g, g, d, t, w1b, b1b, w2b, b2b, w3b, b3b, e, f, lwt, lb2)


def kernel(dense_xyz, dense_feats, nei_inds,
           w1, b1, g1, be1, w2, b2, g2, be2, w3, b3, g3, be3, lw, lb):
    xyz = dense_xyz[0]
    feats = dense_feats[0]
    idx_flat = nei_inds[0].reshape(-1).astype(jnp.int32)
    table = jnp.concatenate(
        [feats, xyz, jnp.zeros((N, ROW - C_IN - 3), jnp.float32)], axis=1)

    grows = _sc_gather(table, idx_flat)
    g4d = grows.reshape(N, 4, 1, 128)

    inv = 1.0 / jnp.sqrt(1.0 + 1e-5)
    s1, s2, s3 = g1 * inv, g2 * inv, g3 * inv
    w1e = w1.T * s1[None, :]
    c1 = b1 * s1 + be1
    w2e = w2.T * s2[None, :]
    c2 = b2 * s2 + be2
    w3e = w3.T * s3[None, :]
    c3 = b3 * s3 + be3

    eyeK = jnp.eye(K, dtype=jnp.float32)
    w1b = jnp.kron(eyeK, w1e)
    w2b = jnp.kron(eyeK, w2e)
    w3b = jnp.kron(eyeK, w3e)
    b1b = jnp.tile(c1, K)[None, :]
    b2b = jnp.tile(c2, K)[None, :]
    b3b = jnp.tile(c3, K)[None, :]

    dk = jnp.zeros((ROW, 3), jnp.float32).at[C_IN:C_IN + 3, :].set(
        jnp.eye(3, dtype=jnp.float32))
    d = jnp.kron(eyeK, dk)
    t = jnp.tile(jnp.eye(3, dtype=jnp.float32), (1, K))
    e = jnp.kron(jnp.eye(C_IN, dtype=jnp.float32),
                 jnp.ones((1, C_MID), jnp.float32)
                 ).astype(jnp.bfloat16)
    f = jnp.tile(jnp.eye(C_MID, dtype=jnp.float32),
                 (1, C_IN)).astype(jnp.bfloat16)

    local48, new_feat = _tc_dense(xyz, g4d, d, t,
                                  w1b, b1b, w2b, b2b, w3b, b3b,
                                  e, f, lw.T, lb[None, :])
    return (new_feat[None], local48.reshape(1, N, K, 3))


# speedup vs baseline: 1.2483x; 1.0213x over previous
import functools

import jax
import jax.numpy as jnp
from jax import lax
from jax.experimental import pallas as pl
from jax.experimental.pallas import tpu as pltpu
from jax.experimental.pallas import tpu_sc as plsc

N = 100000
K = 16
C_IN = 16
C_OUT = 64
C_MID = 16
ROW = 32

NK = N * K
NUM_WORKERS = 32
PER_W = NK // NUM_WORKERS
CHUNK = 1000
NCHUNK = PER_W // CHUNK

B = 2000
GRID = N // B


def _sc_gather(table, idx_flat):
    mesh = plsc.VectorSubcoreMesh(core_axis_name="c", subcore_axis_name="s")

    @functools.partial(
        pl.kernel,
        mesh=mesh,
        out_type=jax.ShapeDtypeStruct((NK, ROW), jnp.float32),
        scratch_types=[
            pltpu.VMEM((CHUNK,), jnp.int32),
            pltpu.VMEM((CHUNK,), jnp.int32),
            pltpu.VMEM((CHUNK, ROW), jnp.float32),
            pltpu.VMEM((CHUNK, ROW), jnp.float32),
            pltpu.SemaphoreType.DMA,
            pltpu.SemaphoreType.DMA,
            pltpu.SemaphoreType.DMA,
            pltpu.SemaphoreType.DMA,
        ],
        compiler_params=pltpu.CompilerParams(use_tc_tiling_on_sc=False),
    )
    def gather_kernel(table_hbm, idx_hbm, out_hbm,
                      idx0, idx1, rows0, rows1, gsem0, gsem1, wsem0, wsem1):
        wid = lax.axis_index("s") * 2 + lax.axis_index("c")
        base = pl.multiple_of(wid * PER_W, 8)
        idx_b = (idx0, idx1)
        rows_b = (rows0, rows1)
        gsem_b = (gsem0, gsem1)
        wsem_b = (wsem0, wsem1)

        for b in range(2):
            pltpu.sync_copy(idx_hbm.at[pl.ds(base + b * CHUNK, CHUNK)],
                            idx_b[b])
            pltpu.async_copy(table_hbm.at[idx_b[b]], rows_b[b], gsem_b[b])

        def body(i, carry):
            for b in range(2):
                j = 2 * i + b
                off = pl.multiple_of(base + j * CHUNK, 8)
                dst = out_hbm.at[pl.ds(off, CHUNK)]
                pltpu.make_async_copy(table_hbm.at[idx_b[b]], rows_b[b],
                                      gsem_b[b]).wait()
                pltpu.async_copy(rows_b[b], dst, wsem_b[b])

                @pl.when(j + 2 < NCHUNK)
                def _prefetch():
                    off2 = pl.multiple_of(off + 2 * CHUNK, 8)
                    pltpu.sync_copy(idx_hbm.at[pl.ds(off2, CHUNK)], idx_b[b])
                    pltpu.make_async_copy(rows_b[b], dst, wsem_b[b]).wait()
                    pltpu.async_copy(table_hbm.at[idx_b[b]], rows_b[b],
                                     gsem_b[b])

                @pl.when(j + 2 >= NCHUNK)
                def _drain():
                    pltpu.make_async_copy(rows_b[b], dst, wsem_b[b]).wait()
            return carry

        lax.fori_loop(0, NCHUNK // 2, body, 0)

    return gather_kernel(table, idx_flat)


def _tc_body(xyz_ref, g0_ref, g1_ref, g2_ref, g3_ref,
             d_ref, t_ref, w1_ref, b1_ref, w2_ref, b2_ref, w3_ref, b3_ref,
             e_ref, f_ref, lwt_ref, lb_ref,
             local_ref, out_ref):
    x = xyz_ref[...]
    f32 = jnp.float32
    qs = [r[:, 0, 0, :] for r in (g0_ref, g1_ref, g2_ref, g3_ref)]
    acc = -jnp.dot(x, t_ref[...], preferred_element_type=f32)
    for q in range(4):
        acc = acc + jnp.dot(qs[q], d_ref[q * 128:(q + 1) * 128, :],
                            preferred_element_type=f32)
    local = acc
    local_ref[...] = local

    h1 = jnp.maximum(jnp.dot(local, w1_ref[...], preferred_element_type=f32)
                     + b1_ref[...], 0.0)
    h2 = jnp.dot(h1, w2_ref[...], preferred_element_type=f32) + b2_ref[...]
    w = jnp.maximum(jnp.dot(h2, w3_ref[...], preferred_element_type=f32)
                    + b3_ref[...], 0.0)
    w_bf = w.astype(jnp.bfloat16)
    qs_bf = [q.astype(jnp.bfloat16) for q in qs]

    e = e_ref[...]
    f = f_ref[...]
    pre = jnp.zeros((B, C_IN * C_MID), f32)
    for k in range(K):
        gfk = qs_bf[k // 4][:, (k % 4) * ROW:(k % 4) * ROW + C_IN]
        wk = w_bf[:, k * C_MID:(k + 1) * C_MID]
        pre = pre + (jnp.dot(gfk, e, preferred_element_type=f32)
                     * jnp.dot(wk, f, preferred_element_type=f32))

    out = jnp.dot(pre, lwt_ref[...], preferred_element_type=f32) + lb_ref[...]
    out_ref[...] = jnp.maximum(out, 0.0)


def _tc_dense(xyz, g, d, t, w1b, b1b, w2b, b2b, w3b, b3b, e, f, lwt, lb2):
    full = lambda shape: pl.BlockSpec(shape, lambda i: (0, 0))
    row = lambda width: pl.BlockSpec((B, width), lambda i: (i, 0))
    return pl.pallas_call(
        _tc_body,
        grid=(GRID,),
        in_specs=[
            row(3),
            pl.BlockSpec((B, 1, 1, 128), lambda i: (i, 0, 0, 0)),
            pl.BlockSpec((B, 1, 1, 128), lambda i: (i, 1, 0, 0)),
            pl.BlockSpec((B, 1, 1, 128), lambda i: (i, 2, 0, 0)),
            pl.BlockSpec((B, 1, 1, 128), lambda i: (i, 3, 0, 0)),
            full((ROW * K, 3 * K)), full((3, 3 * K)),
            full((3 * K, 8 * K)), full((1, 8 * K)),
            full((8 * K, 8 * K)), full((1, 8 * K)),
            full((8 * K, C_MID * K)), full((1, C_MID * K)),
            full((C_IN, C_IN * C_MID)), full((C_MID, C_IN * C_MID)),
            full((C_IN * C_MID, C_OUT)), full((1, C_OUT)),
        ],
        out_specs=[
            pl.BlockSpec((B, 3 * K), lambda i: (i, 0)),
            pl.BlockSpec((B, C_OUT), lambda i: (i, 0)),
        ],
        out_shape=[
            jax.ShapeDtypeStruct((N, 3 * K), jnp.float32),
            jax.ShapeDtypeStruct((N, C_OUT), jnp.float32),
        ],
    )(xyz, g, g, g, g, d, t, w1b, b1b, w2b, b2b, w3b, b3b, e, f, lwt, lb2)


def kernel(dense_xyz, dense_feats, nei_inds,
           w1, b1, g1, be1, w2, b2, g2, be2, w3, b3, g3, be3, lw, lb):
    xyz = dense_xyz[0]
    feats = dense_feats[0]
    idx_flat = nei_inds[0].reshape(-1).astype(jnp.int32)
    table = jnp.concatenate(
        [feats, xyz, jnp.zeros((N, ROW - C_IN - 3), jnp.float32)], axis=1)

    grows = _sc_gather(table, idx_flat)
    g4d = grows.reshape(N, 4, 1, 128)

    inv = 1.0 / jnp.sqrt(1.0 + 1e-5)
    s1, s2, s3 = g1 * inv, g2 * inv, g3 * inv
    w1e = w1.T * s1[None, :]
    c1 = b1 * s1 + be1
    w2e = w2.T * s2[None, :]
    c2 = b2 * s2 + be2
    w3e = w3.T * s3[None, :]
    c3 = b3 * s3 + be3

    eyeK = jnp.eye(K, dtype=jnp.float32)
    w1b = jnp.kron(eyeK, w1e)
    w2b = jnp.kron(eyeK, w2e)
    w3b = jnp.kron(eyeK, w3e)
    b1b = jnp.tile(c1, K)[None, :]
    b2b = jnp.tile(c2, K)[None, :]
    b3b = jnp.tile(c3, K)[None, :]

    dk = jnp.zeros((ROW, 3), jnp.float32).at[C_IN:C_IN + 3, :].set(
        jnp.eye(3, dtype=jnp.float32))
    d = jnp.kron(eyeK, dk)
    t = jnp.tile(jnp.eye(3, dtype=jnp.float32), (1, K))
    e = jnp.kron(jnp.eye(C_IN, dtype=jnp.float32),
                 jnp.ones((1, C_MID), jnp.float32)
                 ).astype(jnp.bfloat16)
    f = jnp.tile(jnp.eye(C_MID, dtype=jnp.float32),
                 (1, C_IN)).astype(jnp.bfloat16)

    local48, new_feat = _tc_dense(xyz, g4d, d, t,
                                  w1b, b1b, w2b, b2b, w3b, b3b,
                                  e, f, lw.T, lb[None, :])
    return (new_feat[None], local48.reshape(1, N, K, 3))
